# Initial kernel scaffold; baseline (speedup 1.0000x reference)
import jax
import jax.numpy as jnp
from jax.experimental import pallas as pl


def kernel(x, pos, edge_index, W1, Wr1, b1, W2, Wr2, b2, W3, Wr3, b3, W4, Wr4, b4, W5, Wr5, b5, A1, a1, A2, a2, A3, a3, A4, a4, F, fb):
    raise NotImplementedError("write your pallas kernel here")


# speedup vs baseline: 1.0000x; 1.0000x over previous
import itertools

import jax
import jax.numpy as jnp
from jax.experimental import pallas as pl

KS = 5
K = KS ** 3
COMBOS = list(itertools.product([0, 1], repeat=3))
GRIDS = [32, 16, 8, 4]


def _log_cartesian(pos, src, dst):
    rel = pos[dst] - pos[src]
    u = jnp.sign(rel) * jnp.log1p(30.0 * jnp.abs(rel)) / jnp.log1p(30.0)
    return jnp.clip(0.5 + 0.5 * u, 0.0, 1.0)


def _spline_conv(x, pos, src, dst, emask, W, Wr, b):
    N, cin = x.shape
    u = _log_cartesian(pos, src, dst)
    p = u * (KS - 1)
    i0 = jnp.clip(jnp.floor(p), 0, KS - 2).astype(jnp.int32)
    f = p - i0
    xs = x[src]
    X = jnp.zeros((N * K, cin), x.dtype)
    dsti = dst.astype(jnp.int32)
    for bits in COMBOS:
        idx = jnp.zeros(src.shape[0], jnp.int32)
        coeff = emask
        for d in range(3):
            idx = idx * KS + (i0[:, d] + bits[d])
            coeff = coeff * (f[:, d] if bits[d] else (1.0 - f[:, d]))
        X = X.at[dsti * K + idx].add(coeff[:, None] * xs)
    deg = jnp.maximum(jnp.zeros((N,), x.dtype).at[dst].add(emask), 1.0)
    msg = jnp.einsum('nki,kio->no', X.reshape(N, K, cin), W)
    return msg / deg[:, None] + x @ Wr + b


def _pool_select(x, pos, src, dst, emask, weight, inv, C):
    N = x.shape[0]
    segmax = jax.ops.segment_max(weight, inv, C)
    iswin = weight >= segmax[inv]
    winner = jax.ops.segment_max(jnp.where(iswin, jnp.arange(N), -1), inv, C)
    nx = x[winner]
    cnt = jax.ops.segment_sum(jnp.ones((N,), x.dtype), inv, C)
    npos = jax.ops.segment_sum(pos, inv, C) / cnt[:, None]
    nsrc = inv[src]
    ndst = inv[dst]
    nmask = emask * (nsrc != ndst).astype(x.dtype)
    return nx, npos, nsrc, ndst, nmask


def _compute_structure(pos):
    N = pos.shape[0]
    invs, Cs, valids = [], [], []
    p = pos
    valid = jnp.ones((N,), bool)
    for g in GRIDS:
        v = jnp.clip(jnp.floor(jnp.clip(p, 0.0, 1.0 - 1e-6) * g), 0, g - 1).astype(jnp.int64)
        code = (v[:, 0] * g + v[:, 1]) * g + v[:, 2]
        sentinel = g * g * g
        code = jnp.where(valid, code, sentinel)
        uniq, inv = jnp.unique(code, return_inverse=True, size=N, fill_value=sentinel)
        C = N
        invs.append(inv.astype(jnp.int32))
        Cs.append(C)
        valid = uniq < sentinel
        valids.append(valid)
        cnt = jax.ops.segment_sum(jnp.ones((N,), p.dtype), inv, C)
        p = jax.ops.segment_sum(p, inv, C) / cnt[:, None]
        p = jnp.where(valid[:, None], p, 0.0)
    return invs, Cs, valids


def kernel(x, pos, edge_index, W1, Wr1, b1, W2, Wr2, b2, W3, Wr3, b3, W4, Wr4, b4, W5, Wr5, b5, A1, a1, A2, a2, A3, a3, A4, a4, F, fb):
    src, dst = edge_index[0], edge_index[1]
    invs, Cs, valids = _compute_structure(pos)
    emask = (src != dst).astype(x.dtype)
    h = jax.nn.elu(_spline_conv(x, pos, src, dst, emask, W1, Wr1, b1))
    att = h @ A1 + a1
    h, pos, src, dst, emask = _pool_select(h, pos, src, dst, emask, att[:, 0], invs[0], Cs[0])
    pos = jnp.where(valids[0][:, None], pos, 0.0)
    h = jax.nn.elu(_spline_conv(h, pos, src, dst, emask, W2, Wr2, b2))
    att = h @ A2 + a2
    h, pos, src, dst, emask = _pool_select(h, pos, src, dst, emask, att[:, 0], invs[1], Cs[1])
    pos = jnp.where(valids[1][:, None], pos, 0.0)
    h = jax.nn.elu(_spline_conv(h, pos, src, dst, emask, W3, Wr3, b3))
    att = h @ A3 + a3
    h, pos, src, dst, emask = _pool_select(h, pos, src, dst, emask, att[:, 0], invs[2], Cs[2])
    pos = jnp.where(valids[2][:, None], pos, 0.0)
    h = jax.nn.elu(_spline_conv(h, pos, src, dst, emask, W4, Wr4, b4))
    att = h @ A4 + a4
    h, pos, src, dst, emask = _pool_select(h, pos, src, dst, emask, att[:, 0], invs[3], Cs[3])
    pos = jnp.where(valids[3][:, None], pos, 0.0)
    h = jax.nn.relu(_spline_conv(h, pos, src, dst, emask, W5, Wr5, b5))
    v = jnp.clip(jnp.floor(pos + 0.5), 0, 1).astype(jnp.int32)
    cid = (v[:, 0] * 2 + v[:, 1]) * 2 + v[:, 2]
    cid = jnp.where(valids[3], cid, 8)
    pooled = jax.ops.segment_max(h, cid, 8)
    cnt = jax.ops.segment_sum(jnp.ones((h.shape[0],), h.dtype), cid, 8)
    pooled = jnp.where(cnt[:, None] > 0, pooled, 0.0)
    z = pooled.reshape(1, 8 * 64)
    logits = z @ F + fb
    return jax.nn.log_softmax(logits, axis=1)


# speedup vs baseline: 1.6765x; 1.6765x over previous
import itertools

import jax
import jax.numpy as jnp
from jax.experimental import pallas as pl

KS = 5
K = KS ** 3
COMBOS = list(itertools.product([0, 1], repeat=3))
GRIDS = [32, 16, 8, 4]


def _conv_combine(xflat, deg, h, wflat, wr, b, acol, a0, act):
    N, kc = xflat.shape
    cin = h.shape[1]
    cout = wr.shape[1]
    bn = 200
    grid = (N // bn,)

    def body(x_ref, d_ref, h_ref, w_ref, wr_ref, b_ref, ac_ref, a0_ref,
             hn_ref, att_ref):
        msg = jnp.dot(x_ref[...], w_ref[...],
                      preferred_element_type=jnp.float32)
        deg_c = jnp.maximum(d_ref[...], 1.0)
        pre = msg / deg_c + jnp.dot(h_ref[...], wr_ref[...],
                                    preferred_element_type=jnp.float32)
        pre = pre + b_ref[...]
        if act == "elu":
            hn = jnp.where(pre > 0, pre, jnp.exp(jnp.minimum(pre, 0.0)) - 1.0)
        else:
            hn = jnp.maximum(pre, 0.0)
        hn_ref[...] = hn
        att_ref[...] = jnp.dot(hn, ac_ref[...],
                               preferred_element_type=jnp.float32) + a0_ref[...]

    hn, att = pl.pallas_call(
        body,
        grid=grid,
        in_specs=[pl.BlockSpec((bn, kc), lambda i: (i, 0)),
                  pl.BlockSpec((bn, 1), lambda i: (i, 0)),
                  pl.BlockSpec((bn, cin), lambda i: (i, 0)),
                  pl.BlockSpec((kc, cout), lambda i: (0, 0)),
                  pl.BlockSpec((cin, cout), lambda i: (0, 0)),
                  pl.BlockSpec((1, cout), lambda i: (0, 0)),
                  pl.BlockSpec((cout, 1), lambda i: (0, 0)),
                  pl.BlockSpec((1, 1), lambda i: (0, 0))],
        out_specs=[pl.BlockSpec((bn, cout), lambda i: (i, 0)),
                   pl.BlockSpec((bn, 1), lambda i: (i, 0))],
        out_shape=[jax.ShapeDtypeStruct((N, cout), jnp.float32),
                   jax.ShapeDtypeStruct((N, 1), jnp.float32)],
    )(xflat, deg, h, wflat, wr, b.reshape(1, cout), acol, a0.reshape(1, 1))
    return hn, att[:, 0]


def _log_cart(pos, src, dst):
    rel = pos[dst] - pos[src]
    u = jnp.sign(rel) * jnp.log1p(30.0 * jnp.abs(rel)) / jnp.log1p(30.0)
    return jnp.clip(0.5 + 0.5 * u, 0.0, 1.0)


def _spline_conv(x, pos, src, dst, emask, W, Wr, b, acol, a0, act):
    N, cin = x.shape
    u = _log_cart(pos, src, dst)
    p = u * (KS - 1)
    i0 = jnp.clip(jnp.floor(p), 0, KS - 2).astype(jnp.int32)
    f = p - i0
    xs = x[src]
    X = jnp.zeros((N * K, cin), x.dtype)
    dsti = dst.astype(jnp.int32)
    for bits in COMBOS:
        idx = jnp.zeros(src.shape[0], jnp.int32)
        coeff = emask
        for d in range(3):
            idx = idx * KS + (i0[:, d] + bits[d])
            coeff = coeff * (f[:, d] if bits[d] else (1.0 - f[:, d]))
        X = X.at[dsti * K + idx].add(coeff[:, None] * xs)
    deg = jnp.zeros((N,), x.dtype).at[dst].add(emask)
    wflat = W.reshape(K * cin, W.shape[2])
    return _conv_combine(X.reshape(N, K * cin), deg[:, None], x, wflat,
                         Wr, b, acol, a0, act)


def _pool_select(x, pos, src, dst, emask, weight, inv):
    N = x.shape[0]
    C = N
    segmax = jax.ops.segment_max(weight, inv, C)
    iswin = weight >= segmax[inv]
    winner = jax.ops.segment_max(jnp.where(iswin, jnp.arange(N), -1), inv, C)
    nx = x[winner]
    cnt = jax.ops.segment_sum(jnp.ones((N,), x.dtype), inv, C)
    npos = jax.ops.segment_sum(pos, inv, C) / cnt[:, None]
    nsrc = inv[src]
    ndst = inv[dst]
    nmask = emask * (nsrc != ndst).astype(x.dtype)
    return nx, npos, nsrc, ndst, nmask


def _structure(pos):
    N = pos.shape[0]
    invs, valids = [], []
    p = pos
    valid = jnp.ones((N,), bool)
    for g in GRIDS:
        v = jnp.clip(jnp.floor(jnp.clip(p, 0.0, 1.0 - 1e-6) * g), 0, g - 1).astype(jnp.int32)
        code = (v[:, 0] * g + v[:, 1]) * g + v[:, 2]
        sentinel = g * g * g
        code = jnp.where(valid, code, sentinel)
        uniq, inv = jnp.unique(code, return_inverse=True, size=N,
                               fill_value=sentinel)
        invs.append(inv.astype(jnp.int32))
        valid = uniq < sentinel
        valids.append(valid)
        cnt = jax.ops.segment_sum(jnp.ones((N,), p.dtype), inv, N)
        p = jax.ops.segment_sum(p, inv, N) / cnt[:, None]
        p = jnp.where(valid[:, None], p, 0.0)
    return invs, valids


def kernel(x, pos, edge_index, W1, Wr1, b1, W2, Wr2, b2, W3, Wr3, b3,
           W4, Wr4, b4, W5, Wr5, b5, A1, a1, A2, a2, A3, a3, A4, a4, F, fb):
    N = x.shape[0]
    src = edge_index[0].astype(jnp.int32)
    dst = edge_index[1].astype(jnp.int32)
    em = (edge_index[0] != edge_index[1]).astype(jnp.float32)

    invs, valids = _structure(pos)

    convs = [(W1, Wr1, b1, A1, a1), (W2, Wr2, b2, A2, a2),
             (W3, Wr3, b3, A3, a3), (W4, Wr4, b4, A4, a4)]
    h = x
    for li, (W, Wr, b, A, a) in enumerate(convs):
        h, att = _spline_conv(h, pos, src, dst, em, W, Wr, b,
                              A[:, 0:1], a[0:1], "elu")
        h, pos, src, dst, em = _pool_select(h, pos, src, dst, em, att,
                                            invs[li])
        pos = jnp.where(valids[li][:, None], pos, 0.0)

    dummy_a = jnp.zeros((64, 1), jnp.float32)
    h, _ = _spline_conv(h, pos, src, dst, em, W5, Wr5, b5,
                        dummy_a, jnp.zeros((1,), jnp.float32), "relu")

    v = jnp.clip(jnp.floor(pos + 0.5), 0, 1).astype(jnp.int32)
    cid = (v[:, 0] * 2 + v[:, 1]) * 2 + v[:, 2]
    cid = jnp.where(valids[3], cid, 8)
    pooled = jax.ops.segment_max(h, cid, 8)
    cnt = jax.ops.segment_sum(jnp.ones((N,), h.dtype), cid, 8)
    pooled = jnp.where(cnt[:, None] > 0, pooled, 0.0)
    z = pooled.reshape(1, 8 * 64)
    logits = z @ F + fb
    return jax.nn.log_softmax(logits, axis=1)
